# baseline (speedup 1.0000x reference)
import functools

import jax
import jax.numpy as jnp
from jax import lax
from jax.experimental import pallas as pl
from jax.experimental.pallas import tpu as pltpu
from jax.experimental.pallas import tpu_sc as plsc

_CH = 256
_NCHUNK = 13
_WAVE = 2048
_EW = 128


@functools.cache
def _make_kernel(B, V, D):
    info = plsc.get_sparse_core_info()
    L = info.num_lanes
    NC = info.num_cores
    NW = NC * info.num_subcores
    n_waves = B // _WAVE
    v_pad_lines = ((V + 127) // 128) * 128
    c0_max = v_pad_lines - _CH
    out_rows = B + 64
    mesh = plsc.VectorSubcoreMesh(core_axis_name="c", subcore_axis_name="s")

    @functools.partial(
        pl.kernel,
        mesh=mesh,
        compiler_params=pltpu.CompilerParams(needs_layout_passes=False),
        out_type=jax.ShapeDtypeStruct((out_rows, 2 * D), jnp.float32),
        scratch_types=[
            pltpu.VMEM((B,), jnp.int32),
            pltpu.VMEM((B,), jnp.int32),
            pltpu.VMEM((B,), jnp.int32),
            pltpu.VMEM((_WAVE,), jnp.int32),
            pltpu.VMEM((_WAVE,), jnp.int32),
            pltpu.VMEM((D, _CH), jnp.float32),
            pltpu.VMEM((_EW, 2 * D), jnp.float32),
            pltpu.VMEM((_EW,), jnp.int32),
            pltpu.SMEM((n_waves + 2,), jnp.int32),
            pltpu.SemaphoreType.DMA,
        ],
    )
    def k(idx_hbm, tt_hbm, out_hbm, idx_v, l1c, l1p, l2c, l2p, chunk, stage,
          posw, cnt, sem):
        wid = lax.axis_index("s") * NC + lax.axis_index("c")
        lanes = lax.iota(jnp.int32, L)
        lo = wid * (_NCHUNK * _CH)
        hi = lo + _NCHUNK * _CH

        pltpu.sync_copy(idx_hbm, idx_v)

        def wave_scan(w, carry):
            def grp(g, off):
                p0 = w * _WAVE + g * L
                v = idx_v[pl.ds(p0, L)]
                m = (v >= lo) & (v < hi)
                cs = plsc.cumsum(m.astype(jnp.int32))
                slot = off + cs - 1
                plsc.store_scatter(l1c, [slot], v, mask=m)
                plsc.store_scatter(l1p, [slot], p0 + lanes, mask=m)
                return off + cs[L - 1]

            off_end = lax.fori_loop(0, _WAVE // L, grp, w * _WAVE)
            cnt[w] = off_end - w * _WAVE
            return carry

        lax.fori_loop(0, n_waves, wave_scan, 0)

        def do_chunk(c, carry):
            cid = wid * _NCHUNK + c
            c0 = cid * _CH
            c0c = jnp.minimum(c0, c0_max)
            copies = []
            for b in range(D // 8):
                copies.append(
                    pltpu.async_copy(
                        tt_hbm.at[pl.ds(b * 8, 8), pl.ds(c0c, _CH)],
                        chunk.at[pl.ds(b * 8, 8), :],
                        sem,
                    )
                )
            for cp in copies:
                cp.wait()

            def do_wave(w, carry2):
                c1 = cnt[w]

                def grp2(g, off):
                    q0 = w * _WAVE + g * L
                    vc = l1c[pl.ds(q0, L)]
                    vp = l1p[pl.ds(q0, L)]
                    valid = (g * L + lanes) < c1
                    m = valid & (vc >= c0) & (vc < c0 + _CH)
                    cs = plsc.cumsum(m.astype(jnp.int32))
                    slot = off + cs - 1
                    plsc.store_scatter(l2c, [slot], vc, mask=m)
                    plsc.store_scatter(l2p, [slot], vp, mask=m)
                    return off + cs[L - 1]

                c2 = lax.fori_loop(0, (c1 + L - 1) // L, grp2, 0)

                def ewave(e, carry3):
                    s0 = e * _EW

                    def egrp(g, carry4):
                        q = s0 + g * L
                        vc = l2c[pl.ds(q, L)]
                        vp = l2p[pl.ds(q, L)]
                        valid = (q + lanes) < c2
                        rc = jnp.where(valid, vc - c0c, 0)
                        pos = jnp.where(valid, vp, B)
                        posw[pl.ds(g * L, L)] = pos
                        slot = g * L + lanes
                        for d in range(D):
                            dv = jnp.full((L,), d, jnp.int32)
                            word = plsc.load_gather(chunk, [dv, rc])
                            plsc.store_scatter(stage, [slot, dv], word)
                        return carry4

                    lax.fori_loop(0, _EW // L, egrp, 0)
                    pltpu.sync_copy(stage, out_hbm.at[posw])
                    return carry3

                lax.fori_loop(0, (c2 + _EW - 1) // _EW, ewave, 0)
                return carry2

            lax.fori_loop(0, n_waves, do_wave, 0)
            return carry

        lax.fori_loop(0, _NCHUNK, do_chunk, 0)

    return k


def kernel(x, table):
    B = x.shape[0]
    V, D = table.shape
    out_k = _make_kernel(B, V, D)(x.astype(jnp.int32), table.T)
    return out_k[:B, :D]


# speedup vs baseline: 95.5219x; 95.5219x over previous
import functools

import jax
import jax.numpy as jnp
from jax import lax
from jax.experimental import pallas as pl
from jax.experimental.pallas import tpu as pltpu
from jax.experimental.pallas import tpu_sc as plsc

_CH = 256
_NCHUNK = 13
_WAVE = 2048
_EW = 128


@functools.cache
def _make_kernel(B, V, D):
    info = plsc.get_sparse_core_info()
    L = info.num_lanes
    NC = info.num_cores
    NW = NC * info.num_subcores
    n_waves = B // _WAVE
    v_pad_lines = ((V + 127) // 128) * 128
    c0_max = v_pad_lines - _CH
    out_rows = B + 64
    mesh = plsc.VectorSubcoreMesh(core_axis_name="c", subcore_axis_name="s")

    @functools.partial(
        pl.kernel,
        mesh=mesh,
        compiler_params=pltpu.CompilerParams(needs_layout_passes=False),
        out_type=jax.ShapeDtypeStruct((out_rows, 2 * D), jnp.float32),
        scratch_types=[
            pltpu.VMEM((B,), jnp.int32),
            pltpu.VMEM((B,), jnp.int32),
            pltpu.VMEM((B,), jnp.int32),
            pltpu.VMEM((B,), jnp.int32),
            pltpu.VMEM((B,), jnp.int32),
            pltpu.VMEM((D, _CH), jnp.float32),
            pltpu.VMEM((_EW, 2 * D), jnp.float32),
            pltpu.VMEM((_EW,), jnp.int32),
            pltpu.SMEM((n_waves + 2,), jnp.int32),
            pltpu.SemaphoreType.DMA,
        ],
    )
    def k(idx_hbm, tt_hbm, out_hbm, idx_v, l1c, l1p, l2c, l2p, chunk, stage,
          posw, cnt, sem):
        wid = lax.axis_index("s") * NC + lax.axis_index("c")
        lanes = lax.iota(jnp.int32, L)
        lo = wid * (_NCHUNK * _CH)
        hi = lo + _NCHUNK * _CH

        pltpu.sync_copy(idx_hbm, idx_v)

        def wave_scan(w, carry):
            def grp(g, off):
                p0 = w * _WAVE + g * L
                v = idx_v[pl.ds(p0, L)]
                m = (v >= lo) & (v < hi)
                cs = plsc.cumsum(m.astype(jnp.int32))
                slot = off + cs - 1
                plsc.store_scatter(l1c, [slot], v, mask=m)
                plsc.store_scatter(l1p, [slot], p0 + lanes, mask=m)
                return off + cs[L - 1]

            off_end = lax.fori_loop(0, _WAVE // L, grp, w * _WAVE)
            cnt[w] = off_end - w * _WAVE
            return carry

        lax.fori_loop(0, n_waves, wave_scan, 0)

        def do_chunk(c, carry):
            cid = wid * _NCHUNK + c
            c0 = cid * _CH
            c0c = jnp.minimum(c0, c0_max)
            copies = []
            for b in range(D // 8):
                copies.append(
                    pltpu.async_copy(
                        tt_hbm.at[pl.ds(b * 8, 8), pl.ds(c0c, _CH)],
                        chunk.at[pl.ds(b * 8, 8), :],
                        sem,
                    )
                )
            for cp in copies:
                cp.wait()

            def wave_l2(w, off0):
                c1 = cnt[w]

                def grp2(g, off):
                    q0 = w * _WAVE + g * L
                    vc = l1c[pl.ds(q0, L)]
                    vp = l1p[pl.ds(q0, L)]
                    valid = (g * L + lanes) < c1
                    m = valid & (vc >= c0) & (vc < c0 + _CH)
                    cs = plsc.cumsum(m.astype(jnp.int32))
                    slot = off + cs - 1
                    plsc.store_scatter(l2c, [slot], vc, mask=m)
                    plsc.store_scatter(l2p, [slot], vp, mask=m)
                    return off + cs[L - 1]

                return lax.fori_loop(0, (c1 + L - 1) // L, grp2, off0)

            c2 = lax.fori_loop(0, n_waves, wave_l2, 0)

            def ewave(e, carry3):
                s0 = e * _EW

                def egrp(g, carry4):
                    q = s0 + g * L
                    vc = l2c[pl.ds(q, L)]
                    vp = l2p[pl.ds(q, L)]
                    valid = (q + lanes) < c2
                    rc = jnp.where(valid, vc - c0c, 0)
                    junk = B + ((g * L + lanes) & 63)
                    pos = jnp.where(valid, vp, junk)
                    posw[pl.ds(g * L, L)] = pos
                    slot = g * L + lanes
                    for d in range(D):
                        dv = jnp.full((L,), d, jnp.int32)
                        word = plsc.load_gather(chunk, [dv, rc])
                        plsc.store_scatter(stage, [slot, dv], word)
                    return carry4

                lax.fori_loop(0, _EW // L, egrp, 0)
                pltpu.sync_copy(stage, out_hbm.at[posw])
                return carry3

            lax.fori_loop(0, (c2 + _EW - 1) // _EW, ewave, 0)
            return carry

        lax.fori_loop(0, _NCHUNK, do_chunk, 0)

    return k


def kernel(x, table):
    B = x.shape[0]
    V, D = table.shape
    out_k = _make_kernel(B, V, D)(x.astype(jnp.int32), table.T)
    return out_k[:B, :D]


# speedup vs baseline: 152.8969x; 1.6006x over previous
import functools

import jax
import jax.numpy as jnp
from jax import lax
from jax.experimental import pallas as pl
from jax.experimental.pallas import tpu as pltpu
from jax.experimental.pallas import tpu_sc as plsc

_CH = 256
_NCHUNK = 13
_WAVE = 2048
_EW = 64
_UNROLL = 4


@functools.cache
def _make_kernel(B, V, D):
    info = plsc.get_sparse_core_info()
    L = info.num_lanes
    NC = info.num_cores
    n_waves = B // _WAVE
    v_pad = ((V + 127) // 128) * 128
    c0_max = v_pad - _CH
    out_rows = B + 64
    span = _NCHUNK * _CH
    mesh = plsc.VectorSubcoreMesh(core_axis_name="c", subcore_axis_name="s")

    @functools.partial(
        pl.kernel,
        mesh=mesh,
        compiler_params=pltpu.CompilerParams(needs_layout_passes=False),
        out_type=jax.ShapeDtypeStruct((out_rows, 2 * D), jnp.float32),
        scratch_types=[
            pltpu.VMEM((B,), jnp.int32),
            pltpu.VMEM((B,), jnp.int32),
            pltpu.VMEM((B,), jnp.int32),
            pltpu.VMEM((B,), jnp.int32),
            pltpu.VMEM((B,), jnp.int32),
            pltpu.VMEM((D, _CH), jnp.float32),
            pltpu.VMEM((D, _CH), jnp.float32),
            pltpu.VMEM((_EW, 2 * D), jnp.float32),
            pltpu.VMEM((_EW,), jnp.int32),
            pltpu.SMEM((n_waves + 2,), jnp.int32),
            pltpu.SemaphoreType.DMA,
            pltpu.SemaphoreType.DMA,
        ],
    )
    def k(idx_hbm, tt_hbm, out_hbm, idx_v, l1c, l1p, l2c, l2p, cbufa, cbufb,
          stage, posw, cnt, sema, semb):
        wid = lax.axis_index("s") * NC + lax.axis_index("c")
        lanes = lax.iota(jnp.int32, L)
        lo = wid * span
        hi = lo + span

        pltpu.sync_copy(idx_hbm, idx_v)

        for g in range(_EW // L):
            posw[pl.ds(g * L, L)] = B + ((g * L + lanes) & 63)

        def wave_scan(w, carry):
            def grp(u, off):
                for s in range(_UNROLL):
                    p0 = w * _WAVE + (u * _UNROLL + s) * L
                    v = idx_v[pl.ds(p0, L)]
                    m = (v >= lo) & (v < hi)
                    cs = plsc.cumsum(m.astype(jnp.int32))
                    slot = off + cs - 1
                    plsc.store_scatter(l1c, [slot], v, mask=m)
                    plsc.store_scatter(l1p, [slot], p0 + lanes, mask=m)
                    off = off + cs[L - 1]
                return off

            off_end = lax.fori_loop(0, _WAVE // (L * _UNROLL), grp, w * _WAVE)
            cnt[w] = off_end - w * _WAVE
            return carry

        lax.fori_loop(0, n_waves, wave_scan, 0)

        def start_chunk(c, cbuf, sem):
            c0c = jnp.minimum(lo + c * _CH, c0_max)
            for b in range(D // 8):
                pltpu.async_copy(
                    tt_hbm.at[pl.ds(b * 8, 8), pl.ds(c0c, _CH)],
                    cbuf.at[pl.ds(b * 8, 8), :],
                    sem,
                )

        def wait_chunk(cbuf, sem):
            pltpu.make_async_copy(
                tt_hbm.at[pl.ds(0, D), pl.ds(0, _CH)], cbuf, sem
            ).wait()

        def process_chunk(c, cbuf):
            c0 = lo + c * _CH
            c0c = jnp.minimum(c0, c0_max)

            def wave_l2(w, off0):
                c1 = cnt[w]

                def grp2(g, off):
                    q0 = w * _WAVE + g * L
                    vc = l1c[pl.ds(q0, L)]
                    vp = l1p[pl.ds(q0, L)]
                    valid = (g * L + lanes) < c1
                    m = valid & (vc >= c0) & (vc < c0 + _CH)
                    cs = plsc.cumsum(m.astype(jnp.int32))
                    slot = off + cs - 1
                    plsc.store_scatter(l2c, [slot], vc, mask=m)
                    plsc.store_scatter(l2p, [slot], vp, mask=m)
                    return off + cs[L - 1]

                return lax.fori_loop(0, (c1 + L - 1) // L, grp2, off0)

            c2 = lax.fori_loop(0, n_waves, wave_l2, 0)

            def ewave(e, carry3):
                s0 = e * _EW
                ngrp = (jnp.minimum(c2 - s0, _EW) + L - 1) // L

                def egrp(g, carry4):
                    q = s0 + g * L
                    vc = l2c[pl.ds(q, L)]
                    vp = l2p[pl.ds(q, L)]
                    valid = (q + lanes) < c2
                    rc = jnp.where(valid, vc - c0c, 0)
                    junk = B + ((g * L + lanes) & 63)
                    pos = jnp.where(valid, vp, junk)
                    posw[pl.ds(g * L, L)] = pos
                    slot = g * L + lanes
                    for d in range(D):
                        dv = jnp.full((L,), d, jnp.int32)
                        word = plsc.load_gather(cbuf, [dv, rc])
                        plsc.store_scatter(stage, [slot, dv], word)
                    return carry4

                lax.fori_loop(0, ngrp, egrp, 0)
                pltpu.sync_copy(stage, out_hbm.at[posw])

                def rejunk(g, carry5):
                    posw[pl.ds(g * L, L)] = B + ((g * L + lanes) & 63)
                    return carry5

                lax.fori_loop(0, ngrp, rejunk, 0)
                return carry3

            lax.fori_loop(0, (c2 + _EW - 1) // _EW, ewave, 0)

        start_chunk(0, cbufa, sema)

        def pair_body(i, carry):
            ca = 2 * i
            wait_chunk(cbufa, sema)
            start_chunk(ca + 1, cbufb, semb)
            process_chunk(ca, cbufa)
            wait_chunk(cbufb, semb)
            start_chunk(ca + 2, cbufa, sema)
            process_chunk(ca + 1, cbufb)
            return carry

        lax.fori_loop(0, (_NCHUNK - 1) // 2, pair_body, 0)
        wait_chunk(cbufa, sema)
        process_chunk(_NCHUNK - 1, cbufa)

    return k


def kernel(x, table):
    B = x.shape[0]
    V, D = table.shape
    out_k = _make_kernel(B, V, D)(x.astype(jnp.int32), table.T)
    return out_k[:B, :D]


# speedup vs baseline: 157.7084x; 1.0315x over previous
import functools

import jax
import jax.numpy as jnp
from jax import lax
from jax.experimental import pallas as pl
from jax.experimental.pallas import tpu as pltpu
from jax.experimental.pallas import tpu_sc as plsc

_CH = 256
_NCHUNK = 13
_WAVE = 2048
_EW = 48
_UNROLL = 4


@functools.cache
def _make_kernel(B, V, D):
    info = plsc.get_sparse_core_info()
    L = info.num_lanes
    NC = info.num_cores
    n_waves = B // _WAVE
    v_pad = ((V + 127) // 128) * 128
    c0_max = v_pad - _CH
    out_rows = B + 64
    span = _NCHUNK * _CH
    mesh = plsc.VectorSubcoreMesh(core_axis_name="c", subcore_axis_name="s")

    @functools.partial(
        pl.kernel,
        mesh=mesh,
        compiler_params=pltpu.CompilerParams(needs_layout_passes=False),
        out_type=jax.ShapeDtypeStruct((out_rows, 2 * D), jnp.float32),
        scratch_types=[
            pltpu.VMEM((B,), jnp.int32),
            pltpu.VMEM((B,), jnp.int32),
            pltpu.VMEM((B,), jnp.int32),
            pltpu.VMEM((B,), jnp.int32),
            pltpu.VMEM((B,), jnp.int32),
            pltpu.VMEM((D, _CH), jnp.float32),
            pltpu.VMEM((D, _CH), jnp.float32),
            pltpu.VMEM((_EW, 2 * D), jnp.float32),
            pltpu.VMEM((_EW, 2 * D), jnp.float32),
            pltpu.VMEM((_EW,), jnp.int32),
            pltpu.VMEM((_EW,), jnp.int32),
            pltpu.SMEM((n_waves + 2,), jnp.int32),
            pltpu.SemaphoreType.DMA,
            pltpu.SemaphoreType.DMA,
            pltpu.SemaphoreType.DMA,
            pltpu.SemaphoreType.DMA,
        ],
    )
    def k(idx_hbm, tt_hbm, out_hbm, idx_v, l1c, l1p, l2c, l2p, cbufa, cbufb,
          stga, stgb, poswa, poswb, cnt, sema, semb, semoa, semob):
        wid = lax.axis_index("s") * NC + lax.axis_index("c")
        lanes = lax.iota(jnp.int32, L)
        lo = wid * span
        hi = lo + span

        pltpu.sync_copy(idx_hbm, idx_v)

        for g in range(_EW // L):
            poswa[pl.ds(g * L, L)] = B + ((g * L + lanes) & 63)
            poswb[pl.ds(g * L, L)] = B + ((g * L + lanes) & 63)

        def wave_scan(w, carry):
            def grp(u, off):
                for s in range(_UNROLL):
                    p0 = w * _WAVE + (u * _UNROLL + s) * L
                    v = idx_v[pl.ds(p0, L)]
                    m = (v >= lo) & (v < hi)
                    cs = plsc.cumsum(m.astype(jnp.int32))
                    slot = off + cs - 1
                    plsc.store_scatter(l1c, [slot], v, mask=m)
                    plsc.store_scatter(l1p, [slot], p0 + lanes, mask=m)
                    off = off + cs[L - 1]
                return off

            off_end = lax.fori_loop(0, _WAVE // (L * _UNROLL), grp, w * _WAVE)
            cnt[w] = off_end - w * _WAVE
            return carry

        def start_chunk(c, cbuf, sem):
            c0c = jnp.minimum(lo + c * _CH, c0_max)
            for b in range(D // 8):
                pltpu.async_copy(
                    tt_hbm.at[pl.ds(b * 8, 8), pl.ds(c0c, _CH)],
                    cbuf.at[pl.ds(b * 8, 8), :],
                    sem,
                )

        def wait_chunk(cbuf, sem):
            pltpu.make_async_copy(
                tt_hbm.at[pl.ds(0, D), pl.ds(0, _CH)], cbuf, sem
            ).wait()

        pltpu.async_copy(stga, out_hbm.at[poswa], semoa)
        pltpu.async_copy(stgb, out_hbm.at[poswb], semob)
        start_chunk(0, cbufa, sema)

        lax.fori_loop(0, n_waves, wave_scan, 0)

        def process_chunk(c, cbuf, stg, psw, semo):
            c0 = lo + c * _CH
            c0c = jnp.minimum(c0, c0_max)

            def wave_l2(w, off0):
                c1 = cnt[w]

                def grp2(g, off):
                    q0 = w * _WAVE + g * L
                    vc = l1c[pl.ds(q0, L)]
                    vp = l1p[pl.ds(q0, L)]
                    valid = (g * L + lanes) < c1
                    m = valid & (vc >= c0) & (vc < c0 + _CH)
                    cs = plsc.cumsum(m.astype(jnp.int32))
                    slot = off + cs - 1
                    plsc.store_scatter(l2c, [slot], vc, mask=m)
                    plsc.store_scatter(l2p, [slot], vp, mask=m)
                    return off + cs[L - 1]

                return lax.fori_loop(0, (c1 + L - 1) // L, grp2, off0)

            c2 = lax.fori_loop(0, n_waves, wave_l2, 0)

            def ewave(e, carry3):
                s0 = e * _EW
                ngrp = (jnp.minimum(c2 - s0, _EW) + L - 1) // L
                pltpu.make_async_copy(stg, out_hbm.at[psw], semo).wait()

                def egrp(g, carry4):
                    q = s0 + g * L
                    vc = l2c[pl.ds(q, L)]
                    vp = l2p[pl.ds(q, L)]
                    valid = (q + lanes) < c2
                    rc = jnp.where(valid, vc - c0c, 0)
                    junk = B + ((g * L + lanes) & 63)
                    pos = jnp.where(valid, vp, junk)
                    psw[pl.ds(g * L, L)] = pos
                    slot = g * L + lanes
                    for d in range(D):
                        dv = jnp.full((L,), d, jnp.int32)
                        word = plsc.load_gather(cbuf, [dv, rc])
                        plsc.store_scatter(stg, [slot, dv], word)
                    return carry4

                lax.fori_loop(0, ngrp, egrp, 0)
                pltpu.async_copy(stg, out_hbm.at[psw], semo)

                def rejunk(g, carry5):
                    psw[pl.ds(g * L, L)] = B + ((g * L + lanes) & 63)
                    return carry5

                lax.fori_loop(0, ngrp, rejunk, 0)
                return carry3

            lax.fori_loop(0, (c2 + _EW - 1) // _EW, ewave, 0)

        def pair_body(i, carry):
            ca = 2 * i
            wait_chunk(cbufa, sema)
            start_chunk(ca + 1, cbufb, semb)
            process_chunk(ca, cbufa, stga, poswa, semoa)
            wait_chunk(cbufb, semb)
            start_chunk(ca + 2, cbufa, sema)
            process_chunk(ca + 1, cbufb, stgb, poswb, semob)
            return carry

        lax.fori_loop(0, (_NCHUNK - 1) // 2, pair_body, 0)
        wait_chunk(cbufa, sema)
        process_chunk(_NCHUNK - 1, cbufa, stga, poswa, semoa)
        pltpu.make_async_copy(stga, out_hbm.at[poswa], semoa).wait()
        pltpu.make_async_copy(stgb, out_hbm.at[poswb], semob).wait()

    return k


def kernel(x, table):
    B = x.shape[0]
    V, D = table.shape
    out_k = _make_kernel(B, V, D)(x.astype(jnp.int32), table.T)
    return out_k[:B, :D]
